# baseline (speedup 1.0000x reference)
import jax
import jax.numpy as jnp
from jax.experimental import pallas as pl

_S = 512
_C = 96
_N = 2 * _C
_HALF = 10
_INV_BW = 1.0 / 21.0
_G = 8


def _band_kernel(x_ref, band_ref):
    xb = x_ref[...]
    r = jax.lax.broadcasted_iota(jnp.int32, (1, _S, _S), 1)
    c = jax.lax.broadcasted_iota(jnp.int32, (1, _S, _S), 2)
    d = c - r
    in_band = (d >= -_HALF) & (d <= _HALF)
    band_ref[:, 0, :] = jnp.sum(jnp.where(in_band, xb, 0.0), axis=1) * _INV_BW


def _attn_kernel(band_ref, cw_ref, pw_ref, pb_ref, out_ref):
    band = band_ref[...]
    cw = cw_ref[...]
    bp = jnp.pad(band, ((0, 0), (3, 3)))
    attn = cw[:, 0:1] * bp[:, 0:_S]
    for k in range(1, 7):
        attn = attn + cw[:, k:k + 1] * bp[:, k:k + _S]
    pw = pw_ref[...]
    a0 = jnp.dot(pw, attn[:_C], preferred_element_type=jnp.float32)
    a1 = jnp.dot(pw, attn[_C:], preferred_element_type=jnp.float32)
    attn = jnp.concatenate([a0, a1], axis=0) + pb_ref[...]
    m = jnp.max(attn, axis=1, keepdims=True)
    e = jnp.exp(attn - m)
    out_ref[...] = e / jnp.sum(e, axis=1, keepdims=True)


def _copy_sub_kernel(x_ref, attn_ref, y_ref):
    xb = x_ref[...]
    at = attn_ref[...]
    r = jax.lax.broadcasted_iota(jnp.int32, (1, _S, _S), 1)
    c = jax.lax.broadcasted_iota(jnp.int32, (1, _S, _S), 2)
    y_ref[...] = jnp.where(r == c, at * xb, xb)


def kernel(x, conv_w, point_w, point_b):
    b, c, h, w = x.shape
    x3 = x.reshape(_N, _S, _S)

    band3 = pl.pallas_call(
        _band_kernel,
        grid=(_N // _G,),
        in_specs=[pl.BlockSpec((_G, _S, _S), lambda n: (n, 0, 0))],
        out_specs=pl.BlockSpec((_G, 1, _S), lambda n: (n, 0, 0)),
        out_shape=jax.ShapeDtypeStruct((_N, 1, _S), jnp.float32),
    )(x3)

    band = band3.reshape(_N, _S)
    cw = jnp.tile(conv_w.reshape(_C, 7), (2, 1))
    pw = point_w.reshape(_C, _C)
    pb = jnp.tile(point_b.reshape(_C, 1), (2, 1))

    attn = pl.pallas_call(
        _attn_kernel,
        out_shape=jax.ShapeDtypeStruct((_N, _S), jnp.float32),
    )(band, cw, pw, pb)

    at3 = attn.reshape(_N, 1, _S)
    out = pl.pallas_call(
        _copy_sub_kernel,
        grid=(_N // _G,),
        in_specs=[
            pl.BlockSpec((_G, _S, _S), lambda n: (n, 0, 0)),
            pl.BlockSpec((_G, 1, _S), lambda n: (n, 0, 0)),
        ],
        out_specs=pl.BlockSpec((_G, _S, _S), lambda n: (n, 0, 0)),
        out_shape=jax.ShapeDtypeStruct((_N, _S, _S), jnp.float32),
    )(x3, at3)

    return out.reshape(b, c, h, w)
